# baseline (speedup 1.0000x reference)
import functools

import jax
import jax.numpy as jnp
from jax import lax
from jax.experimental import pallas as pl
from jax.experimental.pallas import tpu as pltpu
from jax.experimental.pallas import tpu_sc as plsc


def kernel(scan, row_weight, col_weight, depth_weight):
    d, em, h, w = scan.shape
    info = plsc.get_sparse_core_info()
    nc, ns = info.num_cores, info.num_subcores
    nw = nc * ns
    slabs = d * h
    per_w = slabs // nw
    mesh = plsc.VectorSubcoreMesh(core_axis_name="c", subcore_axis_name="s")

    @functools.partial(
        pl.kernel, mesh=mesh,
        compiler_params=pltpu.CompilerParams(use_tc_tiling_on_sc=False),
        out_type=jax.ShapeDtypeStruct((d, h, w, 192), jnp.float32),
        scratch_types=[
            pltpu.VMEM((2, w, 192), jnp.float32),
            pltpu.VMEM((h, 64), jnp.float32),
            pltpu.VMEM((d, 64), jnp.float32),
            pltpu.SemaphoreType.DMA((2,)),
        ],
    )
    def sc_k(row_hbm, col_hbm, depth_hbm, out_hbm, slab_v, row_v,
             depth_v, sems):
        wid = lax.axis_index("s") * nc + lax.axis_index("c")
        base = wid * per_w
        pltpu.sync_copy(depth_hbm.at[pl.ds(0, d)], depth_v)
        pltpu.sync_copy(col_hbm.at[pl.ds(0, w)], row_v)

        def fill_col(ww, carry):
            for b in range(2):
                for k in range(4):
                    slab_v[b, ww, pl.ds(k * 16, 16)] = row_v[ww, pl.ds(k * 16, 16)]
            return carry

        lax.fori_loop(0, w, fill_col, 0)
        pltpu.sync_copy(row_hbm.at[pl.ds(0, h)], row_v)

        def do_slab(i, carry):
            slab = base + i
            di = slab // h
            hi = slab - di * h
            buf = lax.rem(i, 2)

            @pl.when(i >= 2)
            def _():
                prev = base + i - 2
                pdi = prev // h
                phi = prev - pdi * h
                pltpu.make_async_copy(
                    slab_v.at[buf], out_hbm.at[pdi, phi], sems.at[buf]).wait()

            r = [row_v[hi, pl.ds(k * 16, 16)] for k in range(4)]
            dp = [depth_v[di, pl.ds(k * 16, 16)] for k in range(4)]

            def fill(ww, c2):
                for k in range(4):
                    slab_v[buf, ww, pl.ds(64 + k * 16, 16)] = r[k]
                for k in range(4):
                    slab_v[buf, ww, pl.ds(128 + k * 16, 16)] = dp[k]
                return c2

            lax.fori_loop(0, w, fill, 0)
            pltpu.make_async_copy(
                slab_v.at[buf], out_hbm.at[di, hi], sems.at[buf]).start()
            return carry

        lax.fori_loop(0, per_w, do_slab, 0)

        for j in range(2):
            prev = base + per_w - 2 + j
            pdi = prev // h
            phi = prev - pdi * h
            pltpu.make_async_copy(
                slab_v.at[(per_w - 2 + j) % 2], out_hbm.at[pdi, phi],
                sems.at[(per_w - 2 + j) % 2]).wait()

    return sc_k(row_weight, col_weight, depth_weight)
